# baseline (speedup 1.0000x reference)
import functools

import jax
import jax.numpy as jnp
from jax.experimental import pallas as pl

FRACTION = 0.33
INT_MIN = -2147483648
INT_MAX = 2147483647


def _monotone_i32(x):
    b = jax.lax.bitcast_convert_type(x, jnp.int32)
    return jnp.where(b >= 0, b, INT_MIN - b)


def _monotone_to_f32(m):
    b = jnp.where(m >= 0, m, INT_MIN - m)
    return jax.lax.bitcast_convert_type(b, jnp.float32)


def _count_true(mask, use_mxu):
    if use_mxu:
        mb = jnp.where(mask, jnp.bfloat16(1), jnp.bfloat16(0))
        ones = jnp.ones((8, mask.shape[0]), jnp.bfloat16)
        cnt = jax.lax.dot_general(ones, mb, (((1,), (0,)), ((), ())),
                                  preferred_element_type=jnp.float32)
        return cnt[0:1, :].astype(jnp.int32)
    return jnp.sum(mask.astype(jnp.int32), axis=0, keepdims=True)


def _bisect16(vals16, kk, red_shape, use_mxu):
    lo0 = jnp.full(red_shape, -32768, jnp.int32)
    hi0 = jnp.full(red_shape, 32767, jnp.int32)

    def body(_, carry):
        lo, hi = carry
        mid = (lo + hi) >> 1
        cnt = _count_true(vals16 >= mid.astype(jnp.int16), use_mxu)
        pred = cnt >= kk
        return jnp.where(pred, mid, lo), jnp.where(pred, hi, mid)

    lo, _ = jax.lax.fori_loop(0, 16, body, (lo0, hi0))
    return lo


def _kth_largest_m(m, kk, use_mxu=False):
    red_shape = (1, m.shape[1])
    mhi = (m >> 16).astype(jnp.int16)
    h_star = _bisect16(mhi, kk, red_shape, use_mxu)
    h16 = h_star.astype(jnp.int16)
    c_hi = _count_true(mhi > h16, use_mxu)
    r2 = kk - c_hi
    ml = jnp.where(mhi == h16,
                   ((m & 0xFFFF) - 32768).astype(jnp.int16),
                   jnp.int16(-32768))
    c_top = _count_true(ml >= jnp.int16(32767), use_mxu)
    lo2 = _bisect16(ml, r2, red_shape, use_mxu)
    low = jnp.where(c_top >= r2, jnp.int32(32767), lo2)
    return (h_star << 16) | (low + 32768)


def _qkv_kernel(x_ref, w_ref, o_ref):
    o_ref[...] = jnp.dot(x_ref[...], w_ref[...],
                         preferred_element_type=jnp.float32)


def _select_kernel(k_ref, q_ref, sel_ref, *, l_q):
    kv = k_ref[0]
    L = kv.shape[0]
    kk = jnp.int32(l_q)

    m = _monotone_i32(kv)
    t_m = _kth_largest_m(m, kk, use_mxu=True)
    t = _monotone_to_f32(t_m)
    s = jnp.sum(jnp.maximum(kv - t, 0.0), axis=0, keepdims=True)
    k_reduce = s / jnp.float32(l_q) + t

    qb = q_ref[0].astype(jnp.bfloat16).astype(jnp.float32)
    kb = k_reduce.astype(jnp.bfloat16).astype(jnp.float32)
    sq = jnp.sum(qb * kb, axis=1, keepdims=True)

    m2 = _monotone_i32(sq)
    tau = _kth_largest_m(m2, kk)
    gt = m2 > tau
    eq = m2 == tau
    c_gt = jnp.sum(gt.astype(jnp.int32), axis=0, keepdims=True)
    r = kk - c_gt
    iota = jax.lax.broadcasted_iota(jnp.int32, (L, 1), 0)

    def body(_, carry):
        lo_p, hi_p = carry
        mid = (lo_p + hi_p) >> 1
        cnt = jnp.sum((eq & (iota < mid)).astype(jnp.int32), axis=0,
                      keepdims=True)
        pred = cnt >= r
        return jnp.where(pred, lo_p, mid), jnp.where(pred, mid, hi_p)

    lo_p0 = jnp.zeros((1, 1), jnp.int32)
    hi_p0 = jnp.full((1, 1), L, jnp.int32)
    _, p_star = jax.lax.fori_loop(0, 12, body, (lo_p0, hi_p0))
    sel = gt | (eq & (iota < p_star))
    sel_ref[0] = sel.astype(jnp.float32)


def _attn_kernel(q_ref, k_ref, v_ref, sel_ref, o_ref):
    q = q_ref[0]
    kv = k_ref[0]
    v = v_ref[0]
    d = q.shape[1]
    logits = jax.lax.dot_general(
        q, kv, (((1,), (1,)), ((), ())),
        preferred_element_type=jnp.float32) * (1.0 / jnp.sqrt(jnp.float32(d)))
    mx = jnp.max(logits, axis=1, keepdims=True)
    e = jnp.exp(logits - mx)
    attn = e / jnp.sum(e, axis=1, keepdims=True)
    out = jax.lax.dot_general(
        attn, v, (((1,), (0,)), ((), ())),
        preferred_element_type=jnp.float32)
    mean_v = jnp.mean(v, axis=0, keepdims=True)
    sel = sel_ref[0]
    o_ref[0] = jnp.where(sel > 0.0, out, mean_v)


def kernel(x, Wq, Wk, Wv):
    B, L, D = x.shape
    d_attn = Wq.shape[0]
    d_val = Wv.shape[0]
    l_q = int((1.0 - FRACTION) * L)

    w_all = jnp.concatenate([Wq, Wk, Wv], axis=0).T
    x2 = x.reshape(B * L, D)
    N = w_all.shape[1]
    BM, BN = 1024, 1024
    qkv = pl.pallas_call(
        _qkv_kernel,
        grid=(B * L // BM, N // BN),
        in_specs=[
            pl.BlockSpec((BM, D), lambda i, j: (i, 0)),
            pl.BlockSpec((D, BN), lambda i, j: (0, j)),
        ],
        out_specs=pl.BlockSpec((BM, BN), lambda i, j: (i, j)),
        out_shape=jax.ShapeDtypeStruct((B * L, N), jnp.float32),
    )(x2, w_all)
    q3 = qkv[:, :d_attn].reshape(B, L, d_attn)
    k3 = qkv[:, d_attn:2 * d_attn].reshape(B, L, d_attn)
    v3 = qkv[:, 2 * d_attn:].reshape(B, L, d_val)

    sel = pl.pallas_call(
        functools.partial(_select_kernel, l_q=l_q),
        grid=(B,),
        in_specs=[
            pl.BlockSpec((1, L, d_attn), lambda b: (b, 0, 0)),
            pl.BlockSpec((1, L, d_attn), lambda b: (b, 0, 0)),
        ],
        out_specs=pl.BlockSpec((1, L, 1), lambda b: (b, 0, 0)),
        out_shape=jax.ShapeDtypeStruct((B, L, 1), jnp.float32),
    )(k3, q3)

    BQ = 512
    out = pl.pallas_call(
        _attn_kernel,
        grid=(B, L // BQ),
        in_specs=[
            pl.BlockSpec((1, BQ, d_attn), lambda b, i: (b, i, 0)),
            pl.BlockSpec((1, L, d_attn), lambda b, i: (b, 0, 0)),
            pl.BlockSpec((1, L, d_val), lambda b, i: (b, 0, 0)),
            pl.BlockSpec((1, BQ, 1), lambda b, i: (b, i, 0)),
        ],
        out_specs=pl.BlockSpec((1, BQ, d_val), lambda b, i: (b, i, 0)),
        out_shape=jax.ShapeDtypeStruct((B, L, d_val), jnp.float32),
    )(q3, k3, v3, sel)
    return out


# speedup vs baseline: 1.0562x; 1.0562x over previous
import functools

import jax
import jax.numpy as jnp
from jax.experimental import pallas as pl

FRACTION = 0.33
INT_MIN = -2147483648
INT_MAX = 2147483647


def _monotone_i32(x):
    b = jax.lax.bitcast_convert_type(x, jnp.int32)
    return jnp.where(b >= 0, b, INT_MIN - b)


def _monotone_to_f32(m):
    b = jnp.where(m >= 0, m, INT_MIN - m)
    return jax.lax.bitcast_convert_type(b, jnp.float32)


def _count_true(mask, use_mxu):
    if use_mxu:
        mb = jnp.where(mask, jnp.bfloat16(1), jnp.bfloat16(0))
        ones = jnp.ones((8, mask.shape[0]), jnp.bfloat16)
        cnt = jax.lax.dot_general(ones, mb, (((1,), (0,)), ((), ())),
                                  preferred_element_type=jnp.float32)
        return cnt[0:1, :].astype(jnp.int32)
    return jnp.sum(mask.astype(jnp.int32), axis=0, keepdims=True)


def _bisect16(vals16, kk, red_shape, use_mxu):
    lo0 = jnp.full(red_shape, -32768, jnp.int32)
    hi0 = jnp.full(red_shape, 32767, jnp.int32)

    def body(_, carry):
        lo, hi = carry
        mid = (lo + hi) >> 1
        cnt = _count_true(vals16 >= mid.astype(jnp.int16), use_mxu)
        pred = cnt >= kk
        return jnp.where(pred, mid, lo), jnp.where(pred, hi, mid)

    lo, _ = jax.lax.fori_loop(0, 16, body, (lo0, hi0))
    return lo


def _kth_largest_m(m, kk, use_mxu=False):
    red_shape = (1, m.shape[1])
    mhi = (m >> 16).astype(jnp.int16)
    h_star = _bisect16(mhi, kk, red_shape, use_mxu)
    h16 = h_star.astype(jnp.int16)
    c_hi = _count_true(mhi > h16, use_mxu)
    r2 = kk - c_hi
    ml = jnp.where(mhi == h16,
                   ((m & 0xFFFF) - 32768).astype(jnp.int16),
                   jnp.int16(-32768))
    c_top = _count_true(ml >= jnp.int16(32767), use_mxu)
    lo2 = _bisect16(ml, r2, red_shape, use_mxu)
    low = jnp.where(c_top >= r2, jnp.int32(32767), lo2)
    return (h_star << 16) | (low + 32768)


def _qkv_kernel(x_ref, w_ref, o_ref):
    o_ref[0] = jax.lax.dot_general(
        x_ref[...], w_ref[0], (((1,), (1,)), ((), ())),
        preferred_element_type=jnp.float32)


def _select_kernel(k_ref, q_ref, sel_ref, *, l_q):
    kv = k_ref[0]
    L = kv.shape[0]
    kk = jnp.int32(l_q)

    m = _monotone_i32(kv)
    t_m = _kth_largest_m(m, kk, use_mxu=True)
    t = _monotone_to_f32(t_m)
    s = jnp.sum(jnp.maximum(kv - t, 0.0), axis=0, keepdims=True)
    k_reduce = s / jnp.float32(l_q) + t

    qb = q_ref[0].astype(jnp.bfloat16).astype(jnp.float32)
    kb = k_reduce.astype(jnp.bfloat16).astype(jnp.float32)
    sq = jnp.sum(qb * kb, axis=1, keepdims=True)

    m2 = _monotone_i32(sq)
    tau = _kth_largest_m(m2, kk)
    gt = m2 > tau
    eq = m2 == tau
    c_gt = jnp.sum(gt.astype(jnp.int32), axis=0, keepdims=True)
    r = kk - c_gt
    iota = jax.lax.broadcasted_iota(jnp.int32, (L, 1), 0)

    def body(_, carry):
        lo_p, hi_p = carry
        mid = (lo_p + hi_p) >> 1
        cnt = jnp.sum((eq & (iota < mid)).astype(jnp.int32), axis=0,
                      keepdims=True)
        pred = cnt >= r
        return jnp.where(pred, lo_p, mid), jnp.where(pred, mid, hi_p)

    lo_p0 = jnp.zeros((1, 1), jnp.int32)
    hi_p0 = jnp.full((1, 1), L, jnp.int32)
    _, p_star = jax.lax.fori_loop(0, 12, body, (lo_p0, hi_p0))
    sel = gt | (eq & (iota < p_star))
    sel_ref[0] = sel.astype(jnp.float32)


def _attn_kernel(q_ref, k_ref, v_ref, sel_ref, o_ref):
    q = q_ref[0]
    kv = k_ref[0]
    v = v_ref[0]
    d = q.shape[1]
    logits = jax.lax.dot_general(
        q, kv, (((1,), (1,)), ((), ())),
        preferred_element_type=jnp.float32) * (1.0 / jnp.sqrt(jnp.float32(d)))
    mx = jnp.max(logits, axis=1, keepdims=True)
    e = jnp.exp(logits - mx)
    attn = e / jnp.sum(e, axis=1, keepdims=True)
    out = jax.lax.dot_general(
        attn, v, (((1,), (0,)), ((), ())),
        preferred_element_type=jnp.float32)
    mean_v = jnp.mean(v, axis=0, keepdims=True)
    sel = sel_ref[0]
    o_ref[0] = jnp.where(sel > 0.0, out, mean_v)


def kernel(x, Wq, Wk, Wv):
    B, L, D = x.shape
    d_attn = Wq.shape[0]
    d_val = Wv.shape[0]
    l_q = int((1.0 - FRACTION) * L)

    w_all = jnp.stack([Wq, Wk, Wv], axis=0)
    x2 = x.reshape(B * L, D)
    BM = 1024
    qkv = pl.pallas_call(
        _qkv_kernel,
        grid=(B * L // BM, 3),
        in_specs=[
            pl.BlockSpec((BM, D), lambda i, j: (i, 0)),
            pl.BlockSpec((1, d_attn, D), lambda i, j: (j, 0, 0)),
        ],
        out_specs=pl.BlockSpec((1, BM, d_attn), lambda i, j: (j, i, 0)),
        out_shape=jax.ShapeDtypeStruct((3, B * L, d_attn), jnp.float32),
    )(x2, w_all)
    q3 = qkv[0].reshape(B, L, d_attn)
    k3 = qkv[1].reshape(B, L, d_attn)
    v3 = qkv[2].reshape(B, L, d_val)

    sel = pl.pallas_call(
        functools.partial(_select_kernel, l_q=l_q),
        grid=(B,),
        in_specs=[
            pl.BlockSpec((1, L, d_attn), lambda b: (b, 0, 0)),
            pl.BlockSpec((1, L, d_attn), lambda b: (b, 0, 0)),
        ],
        out_specs=pl.BlockSpec((1, L, 1), lambda b: (b, 0, 0)),
        out_shape=jax.ShapeDtypeStruct((B, L, 1), jnp.float32),
    )(k3, q3)

    BQ = 512
    out = pl.pallas_call(
        _attn_kernel,
        grid=(B, L // BQ),
        in_specs=[
            pl.BlockSpec((1, BQ, d_attn), lambda b, i: (b, i, 0)),
            pl.BlockSpec((1, L, d_attn), lambda b, i: (b, 0, 0)),
            pl.BlockSpec((1, L, d_val), lambda b, i: (b, 0, 0)),
            pl.BlockSpec((1, BQ, 1), lambda b, i: (b, i, 0)),
        ],
        out_specs=pl.BlockSpec((1, BQ, d_val), lambda b, i: (b, i, 0)),
        out_shape=jax.ShapeDtypeStruct((B, L, d_val), jnp.float32),
    )(q3, k3, v3, sel)
    return out


# speedup vs baseline: 1.1775x; 1.1148x over previous
import functools

import jax
import jax.numpy as jnp
from jax.experimental import pallas as pl
from jax.experimental.pallas import tpu as pltpu

FRACTION = 0.33
INT_MIN = -2147483648
INT_MAX = 2147483647


def _monotone_i32(x):
    b = jax.lax.bitcast_convert_type(x, jnp.int32)
    return jnp.where(b >= 0, b, INT_MIN - b)


def _monotone_to_f32(m):
    b = jnp.where(m >= 0, m, INT_MIN - m)
    return jax.lax.bitcast_convert_type(b, jnp.float32)


def _count_true(mask, use_mxu):
    if use_mxu:
        mb = jnp.where(mask, jnp.bfloat16(1), jnp.bfloat16(0))
        ones = jnp.ones((8, mask.shape[0]), jnp.bfloat16)
        cnt = jax.lax.dot_general(ones, mb, (((1,), (0,)), ((), ())),
                                  preferred_element_type=jnp.float32)
        return cnt[0:1, :].astype(jnp.int32)
    return jnp.sum(mask.astype(jnp.int32), axis=0, keepdims=True)


def _bisect16(vals16, kk, red_shape, use_mxu):
    lo0 = jnp.full(red_shape, -32768, jnp.int32)
    hi0 = jnp.full(red_shape, 32767, jnp.int32)

    def body(_, carry):
        lo, hi = carry
        mid = (lo + hi) >> 1
        cnt = _count_true(vals16 >= mid.astype(jnp.int16), use_mxu)
        pred = cnt >= kk
        return jnp.where(pred, mid, lo), jnp.where(pred, hi, mid)

    lo, _ = jax.lax.fori_loop(0, 16, body, (lo0, hi0))
    return lo


def _kth_largest_m(m, kk, use_mxu=False):
    red_shape = (1, m.shape[1])
    mhi = (m >> 16).astype(jnp.int16)
    h_star = _bisect16(mhi, kk, red_shape, use_mxu)
    h16 = h_star.astype(jnp.int16)
    c_hi = _count_true(mhi > h16, use_mxu)
    r2 = kk - c_hi
    ml = jnp.where(mhi == h16,
                   ((m & 0xFFFF) - 32768).astype(jnp.int16),
                   jnp.int16(-32768))
    c_top = _count_true(ml >= jnp.int16(32767), use_mxu)
    lo2 = _bisect16(ml, r2, red_shape, use_mxu)
    low = jnp.where(c_top >= r2, jnp.int32(32767), lo2)
    return (h_star << 16) | (low + 32768)


def _qkv_kernel(x_ref, w_ref, o_ref):
    o_ref[0, 0] = jax.lax.dot_general(
        x_ref[...], w_ref[0], (((1,), (1,)), ((), ())),
        preferred_element_type=jnp.float32)


def _select_body(kv, q, sel_ref, l_q):
    L = kv.shape[0]
    kk = jnp.int32(l_q)

    m = _monotone_i32(kv)
    t_m = _kth_largest_m(m, kk, use_mxu=True)
    t = _monotone_to_f32(t_m)
    s = jnp.sum(jnp.maximum(kv - t, 0.0), axis=0, keepdims=True)
    k_reduce = s / jnp.float32(l_q) + t

    qb = q.astype(jnp.bfloat16).astype(jnp.float32)
    kb = k_reduce.astype(jnp.bfloat16).astype(jnp.float32)
    sq = jnp.sum(qb * kb, axis=1, keepdims=True)

    m2 = _monotone_i32(sq)
    tau = _kth_largest_m(m2, kk)
    gt = m2 > tau
    eq = m2 == tau
    c_gt = jnp.sum(gt.astype(jnp.int32), axis=0, keepdims=True)
    r = kk - c_gt
    iota = jax.lax.broadcasted_iota(jnp.int32, (L, 1), 0)

    def body(_, carry):
        lo_p, hi_p = carry
        mid = (lo_p + hi_p) >> 1
        cnt = jnp.sum((eq & (iota < mid)).astype(jnp.int32), axis=0,
                      keepdims=True)
        pred = cnt >= r
        return jnp.where(pred, lo_p, mid), jnp.where(pred, mid, hi_p)

    lo_p0 = jnp.zeros((1, 1), jnp.int32)
    hi_p0 = jnp.full((1, 1), L, jnp.int32)
    _, p_star = jax.lax.fori_loop(0, 12, body, (lo_p0, hi_p0))
    sel = gt | (eq & (iota < p_star))
    sel_ref[...] = sel.astype(jnp.float32)


def _fused_kernel(qkv_hbm, o_ref, qkv_v, sel_ref, mv_ref, sem, *, l_q, bq):
    b = pl.program_id(0)
    j = pl.program_id(1)

    @pl.when(j == 0)
    def _():
        pltpu.make_async_copy(qkv_hbm.at[b], qkv_v, sem).start()
        pltpu.make_async_copy(qkv_hbm.at[b], qkv_v, sem).wait()
        _select_body(qkv_v[1], qkv_v[0], sel_ref, l_q)
        mv_ref[...] = jnp.mean(qkv_v[2], axis=0, keepdims=True)

    q = qkv_v[0, pl.ds(j * bq, bq), :]
    kv = qkv_v[1]
    v = qkv_v[2]
    d = q.shape[1]
    logits = jax.lax.dot_general(
        q, kv, (((1,), (1,)), ((), ())),
        preferred_element_type=jnp.float32) * (1.0 / jnp.sqrt(jnp.float32(d)))
    mx = jnp.max(logits, axis=1, keepdims=True)
    e = jnp.exp(logits - mx)
    attn = e / jnp.sum(e, axis=1, keepdims=True)
    out = jax.lax.dot_general(
        attn, v, (((1,), (0,)), ((), ())),
        preferred_element_type=jnp.float32)
    sel = sel_ref[pl.ds(j * bq, bq), :]
    o_ref[0] = jnp.where(sel > 0.0, out, mv_ref[...])


def kernel(x, Wq, Wk, Wv):
    B, L, D = x.shape
    d_attn = Wq.shape[0]
    d_val = Wv.shape[0]
    l_q = int((1.0 - FRACTION) * L)

    w_all = jnp.stack([Wq, Wk, Wv], axis=0)
    x2 = x.reshape(B * L, D)
    BM = 1024
    qkv = pl.pallas_call(
        _qkv_kernel,
        grid=(B * L // BM, 3),
        in_specs=[
            pl.BlockSpec((BM, D), lambda i, j: (i, 0)),
            pl.BlockSpec((1, d_attn, D), lambda i, j: (j, 0, 0)),
        ],
        out_specs=pl.BlockSpec((1, 1, BM, d_attn),
                               lambda i, j: (i // (L // BM), j, i % (L // BM), 0)),
        out_shape=jax.ShapeDtypeStruct((B, 3, L, d_attn), jnp.float32),
    )(x2, w_all)

    BQ = 256
    out = pl.pallas_call(
        functools.partial(_fused_kernel, l_q=l_q, bq=BQ),
        grid=(B, L // BQ),
        in_specs=[pl.BlockSpec(memory_space=pl.ANY)],
        out_specs=pl.BlockSpec((1, BQ, d_val), lambda b, i: (b, i, 0)),
        out_shape=jax.ShapeDtypeStruct((B, L, d_val), jnp.float32),
        scratch_shapes=[
            pltpu.VMEM((3, L, d_attn), jnp.float32),
            pltpu.VMEM((L, 1), jnp.float32),
            pltpu.VMEM((1, d_val), jnp.float32),
            pltpu.SemaphoreType.DMA,
        ],
    )(qkv)
    return out
